# baseline (speedup 1.0000x reference)
import functools

import jax
import jax.numpy as jnp
from jax import lax
from jax.experimental import pallas as pl
from jax.experimental.pallas import tpu as pltpu
from jax.experimental.pallas import tpu_sc as plsc

N = 10000
E = 320000
F_IN = 128
H = 300
G = 128

NP = 10240
PW = 128
NPL = 3
BM = 512
NB = NP // BM
DW = 128


def _dinv_block(degp_blk):
    deg = 1.0 + degp_blk[0, :, 0] + degp_blk[1, :, 0]
    return jax.lax.rsqrt(deg)


def _write_planes(u_ref, u):
    u_ref[0] = u[:, 0:PW]
    u_ref[1] = u[:, PW : 2 * PW]
    u_ref[2] = jnp.concatenate(
        [u[:, 2 * PW : H], jnp.zeros((BM, NPL * PW - H), jnp.float32)], axis=1
    )


def _read_su(s_ref, u_ref):
    su = [s_ref[0, p] + s_ref[1, p] + u_ref[p] for p in range(NPL)]
    return jnp.concatenate(su, axis=1)[:, 0:H]


def _tc_a_body(x_ref, wp_ref, bp_ref, wc_ref, degp_ref, u_ref):
    dinv = _dinv_block(degp_ref[...])
    h = jax.nn.relu(
        jnp.dot(x_ref[...], wp_ref[...], preferred_element_type=jnp.float32)
        + bp_ref[...]
    )
    g = dinv[:, None] * h
    u = jnp.dot(g, wc_ref[...], preferred_element_type=jnp.float32)
    _write_planes(u_ref, u)


def _tc_a(x_p, W_pre, b_pre2, W_conv, degp):
    return pl.pallas_call(
        _tc_a_body,
        grid=(NB,),
        in_specs=[
            pl.BlockSpec((BM, F_IN), lambda i: (i, 0)),
            pl.BlockSpec((F_IN, H), lambda i: (0, 0)),
            pl.BlockSpec((1, H), lambda i: (0, 0)),
            pl.BlockSpec((H, H), lambda i: (0, 0)),
            pl.BlockSpec((2, BM, DW), lambda i: (0, i, 0)),
        ],
        out_specs=pl.BlockSpec((NPL, BM, PW), lambda i: (0, i, 0)),
        out_shape=jax.ShapeDtypeStruct((NPL, NP, PW), jnp.float32),
    )(x_p, W_pre, b_pre2, W_conv, degp)


def _tc_c_body(s_ref, u_ref, degp_ref, bc_ref, wc_ref, un_ref):
    dinv = _dinv_block(degp_ref[...])
    su = _read_su(s_ref, u_ref)
    h = jax.nn.relu(dinv[:, None] * su + bc_ref[...])
    g = dinv[:, None] * h
    un = jnp.dot(g, wc_ref[...], preferred_element_type=jnp.float32)
    _write_planes(un_ref, un)


def _tc_c(s, u, degp, b_conv2, W_conv):
    return pl.pallas_call(
        _tc_c_body,
        grid=(NB,),
        in_specs=[
            pl.BlockSpec((2, NPL, BM, PW), lambda i: (0, 0, i, 0)),
            pl.BlockSpec((NPL, BM, PW), lambda i: (0, i, 0)),
            pl.BlockSpec((2, BM, DW), lambda i: (0, i, 0)),
            pl.BlockSpec((1, H), lambda i: (0, 0)),
            pl.BlockSpec((H, H), lambda i: (0, 0)),
        ],
        out_specs=pl.BlockSpec((NPL, BM, PW), lambda i: (0, i, 0)),
        out_shape=jax.ShapeDtypeStruct((NPL, NP, PW), jnp.float32),
    )(s, u, degp, b_conv2, W_conv)


def _tc_d_body(s_ref, u_ref, degp_ref, bc_ref, wr_ref, batch_ref, out_ref, acc):
    i = pl.program_id(0)

    @pl.when(i == 0)
    def _init():
        acc[...] = jnp.zeros_like(acc)

    dinv = _dinv_block(degp_ref[...])
    su = _read_su(s_ref, u_ref)
    h = jax.nn.relu(dinv[:, None] * su + bc_ref[...])
    r = jnp.sum(h * wr_ref[...], axis=1)
    ids = batch_ref[0, 0, :]
    oh = (ids[:, None] == jax.lax.broadcasted_iota(jnp.int32, (1, G), 1)).astype(
        jnp.float32
    )
    acc[0, :] += jnp.sum(oh * r[:, None], axis=0)
    acc[1, :] += jnp.sum(oh, axis=0)

    @pl.when(i == NB - 1)
    def _fin():
        out_ref[...] = (acc[0:1, :] / jnp.maximum(acc[1:2, :], 1.0))


def _tc_d(s, u, degp, b_conv2, W_read2, batch3):
    return pl.pallas_call(
        _tc_d_body,
        grid=(NB,),
        in_specs=[
            pl.BlockSpec((2, NPL, BM, PW), lambda i: (0, 0, i, 0)),
            pl.BlockSpec((NPL, BM, PW), lambda i: (0, i, 0)),
            pl.BlockSpec((2, BM, DW), lambda i: (0, i, 0)),
            pl.BlockSpec((1, H), lambda i: (0, 0)),
            pl.BlockSpec((1, H), lambda i: (0, 0)),
            pl.BlockSpec((1, 1, BM), lambda i: (i, 0, 0)),
        ],
        out_specs=pl.BlockSpec((1, G), lambda i: (0, 0)),
        out_shape=jax.ShapeDtypeStruct((1, G), jnp.float32),
        scratch_shapes=[pltpu.VMEM((2, G), jnp.float32)],
    )(s, u, degp, b_conv2, W_read2, batch3)


_MESH = plsc.VectorSubcoreMesh(core_axis_name="c", subcore_axis_name="s")
NSUB = 16
RPT = NP // NSUB
CH = 80
NCH = 128
EC = NCH * CH
E2 = 32 * EC
NW = 2 * NSUB
NSEG = 8
SEG = NCH // NSEG


@functools.partial(
    pl.kernel,
    mesh=_MESH,
    out_type=jax.ShapeDtypeStruct((2 * NP, DW), jnp.float32),
    scratch_types=[
        pltpu.VMEM((NCH, CH), jnp.int32),
        pltpu.VMEM((CH, DW), jnp.float32),
        pltpu.VMEM((CH, DW), jnp.float32),
        pltpu.VMEM_SHARED((NP, DW), jnp.float32),
    ],
)
def _sc_deg(dst_hbm, out_hbm, dst_buf, ones_v, zeros_v, acc_sh):
    c = lax.axis_index("c")
    t = lax.axis_index("s")
    wid = c * NSUB + t

    def _fill(r, carry):
        for j in range(DW // 16):
            ones_v[r, pl.ds(j * 16, 16)] = jnp.ones((16,), jnp.float32)
            zeros_v[r, pl.ds(j * 16, 16)] = jnp.zeros((16,), jnp.float32)
        return carry

    lax.fori_loop(0, CH, _fill, 0)
    pltpu.sync_copy(dst_hbm.at[pl.ds(wid * NCH, NCH)], dst_buf)
    for b in range(RPT // CH):
        pltpu.sync_copy(zeros_v, acc_sh.at[pl.ds(t * RPT + b * CH, CH)])
    plsc.subcore_barrier()

    def _step(k, carry):
        pltpu.sync_copy(ones_v, acc_sh.at[dst_buf.at[k]], add=True)
        return carry

    lax.fori_loop(0, NCH, _step, 0)
    plsc.subcore_barrier()
    pltpu.sync_copy(acc_sh.at[pl.ds(t * RPT, RPT)],
                    out_hbm.at[pl.ds(c * NP + t * RPT, RPT)])


@functools.partial(
    pl.kernel,
    mesh=_MESH,
    out_type=jax.ShapeDtypeStruct((2 * NPL * NP, PW), jnp.float32),
    scratch_types=[
        pltpu.VMEM((CH,), jnp.int32),
        pltpu.VMEM((CH,), jnp.int32),
        pltpu.VMEM((CH, PW), jnp.float32),
        pltpu.VMEM_SHARED((NP, PW), jnp.float32),
        pltpu.SemaphoreType.DMA,
    ],
)
def _sc_scatter(u_hbm, src_hbm, dst_hbm, out_hbm, psrc0, psrc1, stag0,
                acc_sh, sem0):
    c = lax.axis_index("c")
    t = lax.axis_index("s")
    wid = c * NSUB + t

    def _zrow(r, carry):
        for j in range(PW // 16):
            stag0[r, pl.ds(j * 16, 16)] = jnp.zeros((16,), jnp.float32)
        return carry

    for p in range(NPL):
        lax.fori_loop(0, CH, _zrow, 0)
        for b in range(RPT // CH):
            pltpu.sync_copy(stag0, acc_sh.at[pl.ds(t * RPT + b * CH, CH)])
        plsc.subcore_barrier()

        offv = jnp.broadcast_to(p * NP, (16,)).astype(jnp.int32)

        def _step(k, carry):
            base = wid * EC + k * CH
            pltpu.sync_copy(src_hbm.at[pl.ds(base, CH)], psrc0)
            pltpu.sync_copy(dst_hbm.at[pl.ds(base, CH)], psrc1)
            for j in range(CH // 16):
                psrc0[pl.ds(j * 16, 16)] = psrc0[pl.ds(j * 16, 16)] + offv
            pltpu.async_copy(u_hbm.at[psrc0], stag0, sem0).wait()
            pltpu.sync_copy(stag0, acc_sh.at[psrc1], add=True)
            return carry

        lax.fori_loop(0, NCH, _step, 0)

        plsc.subcore_barrier()
        pltpu.sync_copy(
            acc_sh.at[pl.ds(t * RPT, RPT)],
            out_hbm.at[pl.ds((c * NPL + p) * NP + t * RPT, RPT)],
        )


def kernel(x, edge_index, batch, W_pre, b_pre, W_conv, b_conv, W_read, b_read):
    pad = E2 - E
    src_p = jnp.concatenate([edge_index[0], jnp.zeros((pad,), jnp.int32)])
    dst_pad = N + (jnp.arange(pad, dtype=jnp.int32) % (NP - N))
    dst_p = jnp.concatenate([edge_index[1], dst_pad])
    src4 = src_p.reshape(32 * NCH, CH)
    dst4 = dst_p.reshape(32 * NCH, CH)
    x_p = jnp.pad(x, ((0, NP - N), (0, 0)))
    batch3 = jnp.pad(batch, (0, NP - N), constant_values=G).reshape(NB, 1, BM)
    b_pre2 = b_pre.reshape(1, H)
    b_conv2 = b_conv.reshape(1, H)
    W_read2 = W_read.reshape(1, H)

    degp = _sc_deg(dst4).reshape(2, NP, DW)
    u = _tc_a(x_p, W_pre, b_pre2, W_conv, degp)
    for _ in range(2):
        s = _sc_scatter(u.reshape(NPL * NP, PW), src_p, dst_p).reshape(2, NPL, NP, PW)
        u = _tc_c(s, u, degp, b_conv2, W_conv)
    s = _sc_scatter(u.reshape(NPL * NP, PW), src_p, dst_p).reshape(2, NPL, NP, PW)
    out = _tc_d(s, u, degp, b_conv2, W_read2, batch3)
    return out.reshape(G) + b_read[0]


# speedup vs baseline: 2.0493x; 2.0493x over previous
import functools

import jax
import jax.numpy as jnp
from jax import lax
from jax.experimental import pallas as pl
from jax.experimental.pallas import tpu as pltpu
from jax.experimental.pallas import tpu_sc as plsc

N = 10000
E = 320000
F_IN = 128
H = 300
G = 128

NP = 10240
PW = 128
NPL = 3
BM = 512
NB = NP // BM
DW = 128


def _dinv_block(degp_blk):
    deg = 1.0 + degp_blk[0, :, 0] + degp_blk[1, :, 0]
    return jax.lax.rsqrt(deg)


def _write_planes(u_ref, u):
    u_ref[0] = u[:, 0:PW]
    u_ref[1] = u[:, PW : 2 * PW]
    u_ref[2] = jnp.concatenate(
        [u[:, 2 * PW : H], jnp.zeros((BM, NPL * PW - H), jnp.float32)], axis=1
    )


def _read_su(s_ref, u_ref):
    su = [s_ref[0, p] + s_ref[1, p] + u_ref[p] for p in range(NPL)]
    return jnp.concatenate(su, axis=1)[:, 0:H]


def _tc_a_body(x_ref, wp_ref, bp_ref, wc_ref, degp_ref, u_ref):
    dinv = _dinv_block(degp_ref[...])
    h = jax.nn.relu(
        jnp.dot(x_ref[...], wp_ref[...], preferred_element_type=jnp.float32)
        + bp_ref[...]
    )
    g = dinv[:, None] * h
    u = jnp.dot(g, wc_ref[...], preferred_element_type=jnp.float32)
    _write_planes(u_ref, u)


def _tc_a(x_p, W_pre, b_pre2, W_conv, degp):
    return pl.pallas_call(
        _tc_a_body,
        grid=(NB,),
        in_specs=[
            pl.BlockSpec((BM, F_IN), lambda i: (i, 0)),
            pl.BlockSpec((F_IN, H), lambda i: (0, 0)),
            pl.BlockSpec((1, H), lambda i: (0, 0)),
            pl.BlockSpec((H, H), lambda i: (0, 0)),
            pl.BlockSpec((2, BM, DW), lambda i: (0, i, 0)),
        ],
        out_specs=pl.BlockSpec((NPL, BM, PW), lambda i: (0, i, 0)),
        out_shape=jax.ShapeDtypeStruct((NPL, NP, PW), jnp.float32),
    )(x_p, W_pre, b_pre2, W_conv, degp)


def _tc_c_body(s_ref, u_ref, degp_ref, bc_ref, wc_ref, un_ref):
    dinv = _dinv_block(degp_ref[...])
    su = _read_su(s_ref, u_ref)
    h = jax.nn.relu(dinv[:, None] * su + bc_ref[...])
    g = dinv[:, None] * h
    un = jnp.dot(g, wc_ref[...], preferred_element_type=jnp.float32)
    _write_planes(un_ref, un)


def _tc_c(s, u, degp, b_conv2, W_conv):
    return pl.pallas_call(
        _tc_c_body,
        grid=(NB,),
        in_specs=[
            pl.BlockSpec((2, NPL, BM, PW), lambda i: (0, 0, i, 0)),
            pl.BlockSpec((NPL, BM, PW), lambda i: (0, i, 0)),
            pl.BlockSpec((2, BM, DW), lambda i: (0, i, 0)),
            pl.BlockSpec((1, H), lambda i: (0, 0)),
            pl.BlockSpec((H, H), lambda i: (0, 0)),
        ],
        out_specs=pl.BlockSpec((NPL, BM, PW), lambda i: (0, i, 0)),
        out_shape=jax.ShapeDtypeStruct((NPL, NP, PW), jnp.float32),
    )(s, u, degp, b_conv2, W_conv)


def _tc_d_body(s_ref, u_ref, degp_ref, bc_ref, wr_ref, batch_ref, out_ref, acc):
    i = pl.program_id(0)

    @pl.when(i == 0)
    def _init():
        acc[...] = jnp.zeros_like(acc)

    dinv = _dinv_block(degp_ref[...])
    su = _read_su(s_ref, u_ref)
    h = jax.nn.relu(dinv[:, None] * su + bc_ref[...])
    r = jnp.sum(h * wr_ref[...], axis=1)
    ids = batch_ref[0, 0, :]
    oh = (ids[:, None] == jax.lax.broadcasted_iota(jnp.int32, (1, G), 1)).astype(
        jnp.float32
    )
    acc[0, :] += jnp.sum(oh * r[:, None], axis=0)
    acc[1, :] += jnp.sum(oh, axis=0)

    @pl.when(i == NB - 1)
    def _fin():
        out_ref[...] = (acc[0:1, :] / jnp.maximum(acc[1:2, :], 1.0))


def _tc_d(s, u, degp, b_conv2, W_read2, batch3):
    return pl.pallas_call(
        _tc_d_body,
        grid=(NB,),
        in_specs=[
            pl.BlockSpec((2, NPL, BM, PW), lambda i: (0, 0, i, 0)),
            pl.BlockSpec((NPL, BM, PW), lambda i: (0, i, 0)),
            pl.BlockSpec((2, BM, DW), lambda i: (0, i, 0)),
            pl.BlockSpec((1, H), lambda i: (0, 0)),
            pl.BlockSpec((1, H), lambda i: (0, 0)),
            pl.BlockSpec((1, 1, BM), lambda i: (i, 0, 0)),
        ],
        out_specs=pl.BlockSpec((1, G), lambda i: (0, 0)),
        out_shape=jax.ShapeDtypeStruct((1, G), jnp.float32),
        scratch_shapes=[pltpu.VMEM((2, G), jnp.float32)],
    )(s, u, degp, b_conv2, W_read2, batch3)


_MESH = plsc.VectorSubcoreMesh(core_axis_name="c", subcore_axis_name="s")
NSUB = 16
RPT = NP // NSUB
CH = 80
NCH = 128
EC = NCH * CH
E2 = 32 * EC
NW = 2 * NSUB
NSEG = 8
SEG = NCH // NSEG


@functools.partial(
    pl.kernel,
    mesh=_MESH,
    out_type=jax.ShapeDtypeStruct((2 * NP, DW), jnp.float32),
    scratch_types=[
        pltpu.VMEM((NCH, CH), jnp.int32),
        pltpu.VMEM((CH, DW), jnp.float32),
        pltpu.VMEM((CH, DW), jnp.float32),
        pltpu.VMEM_SHARED((NP, DW), jnp.float32),
    ],
)
def _sc_deg(dst_hbm, out_hbm, dst_buf, ones_v, zeros_v, acc_sh):
    c = lax.axis_index("c")
    t = lax.axis_index("s")
    wid = c * NSUB + t

    def _fill(r, carry):
        for j in range(DW // 16):
            ones_v[r, pl.ds(j * 16, 16)] = jnp.ones((16,), jnp.float32)
            zeros_v[r, pl.ds(j * 16, 16)] = jnp.zeros((16,), jnp.float32)
        return carry

    lax.fori_loop(0, CH, _fill, 0)
    pltpu.sync_copy(dst_hbm.at[pl.ds(wid * NCH, NCH)], dst_buf)
    for b in range(RPT // CH):
        pltpu.sync_copy(zeros_v, acc_sh.at[pl.ds(t * RPT + b * CH, CH)])
    plsc.subcore_barrier()

    def _step(k, carry):
        pltpu.sync_copy(ones_v, acc_sh.at[dst_buf.at[k]], add=True)
        return carry

    lax.fori_loop(0, NCH, _step, 0)
    plsc.subcore_barrier()
    pltpu.sync_copy(acc_sh.at[pl.ds(t * RPT, RPT)],
                    out_hbm.at[pl.ds(c * NP + t * RPT, RPT)])


@functools.partial(
    pl.kernel,
    mesh=_MESH,
    out_type=jax.ShapeDtypeStruct((2 * NPL * NP, PW), jnp.float32),
    scratch_types=[
        pltpu.VMEM((CH,), jnp.int32),
        pltpu.VMEM((CH,), jnp.int32),
        pltpu.VMEM((CH, PW), jnp.float32),
        pltpu.VMEM_SHARED((NP, PW), jnp.float32),
        pltpu.SemaphoreType.DMA,
    ],
)
def _sc_scatter(u_hbm, src_hbm, dst_hbm, out_hbm, psrc0, psrc1, stag0,
                acc_sh, sem0):
    c = lax.axis_index("c")
    t = lax.axis_index("s")
    wid = c * NSUB + t

    def _zrow(r, carry):
        for j in range(PW // 16):
            stag0[r, pl.ds(j * 16, 16)] = jnp.zeros((16,), jnp.float32)
        return carry

    for p in range(NPL):
        lax.fori_loop(0, CH, _zrow, 0)
        for b in range(RPT // CH):
            pltpu.sync_copy(stag0, acc_sh.at[pl.ds(t * RPT + b * CH, CH)])
        plsc.subcore_barrier()

        offv = jnp.broadcast_to(p * NP, (16,)).astype(jnp.int32)

        def _step(k, carry):
            base = wid * EC + k * CH
            pltpu.sync_copy(src_hbm.at[pl.ds(base, CH)], psrc0)
            pltpu.sync_copy(dst_hbm.at[pl.ds(base, CH)], psrc1)
            for j in range(CH // 16):
                psrc0[pl.ds(j * 16, 16)] = psrc0[pl.ds(j * 16, 16)] + offv
            pltpu.async_copy(u_hbm.at[psrc0], stag0, sem0).wait()
            pltpu.sync_copy(stag0, acc_sh.at[psrc1], add=True)
            return carry

        lax.fori_loop(0, NCH, _step, 0)

        plsc.subcore_barrier()
        pltpu.sync_copy(
            acc_sh.at[pl.ds(t * RPT, RPT)],
            out_hbm.at[pl.ds((c * NPL + p) * NP + t * RPT, RPT)],
        )


def kernel(x, edge_index, batch, W_pre, b_pre, W_conv, b_conv, W_read, b_read):
    pad = E2 - E
    src_pad = jnp.arange(pad, dtype=jnp.int32) % N
    src_p = jnp.concatenate([edge_index[0], src_pad])
    dst_pad = N + (jnp.arange(pad, dtype=jnp.int32) % (NP - N))
    dst_p = jnp.concatenate([edge_index[1], dst_pad])
    src4 = src_p.reshape(32 * NCH, CH)
    dst4 = dst_p.reshape(32 * NCH, CH)
    x_p = jnp.pad(x, ((0, NP - N), (0, 0)))
    batch3 = jnp.pad(batch, (0, NP - N), constant_values=G).reshape(NB, 1, BM)
    b_pre2 = b_pre.reshape(1, H)
    b_conv2 = b_conv.reshape(1, H)
    W_read2 = W_read.reshape(1, H)

    degp = _sc_deg(dst4).reshape(2, NP, DW)
    u = _tc_a(x_p, W_pre, b_pre2, W_conv, degp)
    for _ in range(2):
        s = _sc_scatter(u.reshape(NPL * NP, PW), src_p, dst_p).reshape(2, NPL, NP, PW)
        u = _tc_c(s, u, degp, b_conv2, W_conv)
    s = _sc_scatter(u.reshape(NPL * NP, PW), src_p, dst_p).reshape(2, NPL, NP, PW)
    out = _tc_d(s, u, degp, b_conv2, W_read2, batch3)
    return out.reshape(G) + b_read[0]


# speedup vs baseline: 4.1919x; 2.0456x over previous
import functools

import jax
import jax.numpy as jnp
from jax import lax
from jax.experimental import pallas as pl
from jax.experimental.pallas import tpu as pltpu
from jax.experimental.pallas import tpu_sc as plsc

N = 10000
E = 320000
F_IN = 128
H = 300
G = 128

NP = 10240
PW = 128
NPL = 3
BM = 512
NB = NP // BM
DW = 128


def _dinv_block(degp_blk):
    deg = 1.0 + degp_blk[0, :, 0] + degp_blk[1, :, 0]
    return jax.lax.rsqrt(deg)


def _write_planes(u_ref, u):
    u_ref[0] = u[:, 0:PW]
    u_ref[1] = u[:, PW : 2 * PW]
    u_ref[2] = jnp.concatenate(
        [u[:, 2 * PW : H], jnp.zeros((BM, NPL * PW - H), jnp.float32)], axis=1
    )


def _read_su(s_ref, u_ref):
    su = [s_ref[0, p] + s_ref[1, p] + u_ref[p] for p in range(NPL)]
    return jnp.concatenate(su, axis=1)[:, 0:H]


def _tc_a_body(x_ref, wp_ref, bp_ref, wc_ref, degp_ref, u_ref):
    dinv = _dinv_block(degp_ref[...])
    h = jax.nn.relu(
        jnp.dot(x_ref[...], wp_ref[...], preferred_element_type=jnp.float32)
        + bp_ref[...]
    )
    g = dinv[:, None] * h
    u = jnp.dot(g, wc_ref[...], preferred_element_type=jnp.float32)
    _write_planes(u_ref, u)


def _tc_a(x_p, W_pre, b_pre2, W_conv, degp):
    return pl.pallas_call(
        _tc_a_body,
        grid=(NB,),
        in_specs=[
            pl.BlockSpec((BM, F_IN), lambda i: (i, 0)),
            pl.BlockSpec((F_IN, H), lambda i: (0, 0)),
            pl.BlockSpec((1, H), lambda i: (0, 0)),
            pl.BlockSpec((H, H), lambda i: (0, 0)),
            pl.BlockSpec((2, BM, DW), lambda i: (0, i, 0)),
        ],
        out_specs=pl.BlockSpec((NPL, BM, PW), lambda i: (0, i, 0)),
        out_shape=jax.ShapeDtypeStruct((NPL, NP, PW), jnp.float32),
    )(x_p, W_pre, b_pre2, W_conv, degp)


def _tc_c_body(s_ref, u_ref, degp_ref, bc_ref, wc_ref, un_ref):
    dinv = _dinv_block(degp_ref[...])
    su = _read_su(s_ref, u_ref)
    h = jax.nn.relu(dinv[:, None] * su + bc_ref[...])
    g = dinv[:, None] * h
    un = jnp.dot(g, wc_ref[...], preferred_element_type=jnp.float32)
    _write_planes(un_ref, un)


def _tc_c(s, u, degp, b_conv2, W_conv):
    return pl.pallas_call(
        _tc_c_body,
        grid=(NB,),
        in_specs=[
            pl.BlockSpec((2, NPL, BM, PW), lambda i: (0, 0, i, 0)),
            pl.BlockSpec((NPL, BM, PW), lambda i: (0, i, 0)),
            pl.BlockSpec((2, BM, DW), lambda i: (0, i, 0)),
            pl.BlockSpec((1, H), lambda i: (0, 0)),
            pl.BlockSpec((H, H), lambda i: (0, 0)),
        ],
        out_specs=pl.BlockSpec((NPL, BM, PW), lambda i: (0, i, 0)),
        out_shape=jax.ShapeDtypeStruct((NPL, NP, PW), jnp.float32),
    )(s, u, degp, b_conv2, W_conv)


def _tc_d_body(s_ref, u_ref, degp_ref, bc_ref, wr_ref, batch_ref, out_ref, acc):
    i = pl.program_id(0)

    @pl.when(i == 0)
    def _init():
        acc[...] = jnp.zeros_like(acc)

    dinv = _dinv_block(degp_ref[...])
    su = _read_su(s_ref, u_ref)
    h = jax.nn.relu(dinv[:, None] * su + bc_ref[...])
    r = jnp.sum(h * wr_ref[...], axis=1)
    ids = batch_ref[0, 0, :]
    oh = (ids[:, None] == jax.lax.broadcasted_iota(jnp.int32, (1, G), 1)).astype(
        jnp.float32
    )
    acc[0, :] += jnp.sum(oh * r[:, None], axis=0)
    acc[1, :] += jnp.sum(oh, axis=0)

    @pl.when(i == NB - 1)
    def _fin():
        out_ref[...] = (acc[0:1, :] / jnp.maximum(acc[1:2, :], 1.0))


def _tc_d(s, u, degp, b_conv2, W_read2, batch3):
    return pl.pallas_call(
        _tc_d_body,
        grid=(NB,),
        in_specs=[
            pl.BlockSpec((2, NPL, BM, PW), lambda i: (0, 0, i, 0)),
            pl.BlockSpec((NPL, BM, PW), lambda i: (0, i, 0)),
            pl.BlockSpec((2, BM, DW), lambda i: (0, i, 0)),
            pl.BlockSpec((1, H), lambda i: (0, 0)),
            pl.BlockSpec((1, H), lambda i: (0, 0)),
            pl.BlockSpec((1, 1, BM), lambda i: (i, 0, 0)),
        ],
        out_specs=pl.BlockSpec((1, G), lambda i: (0, 0)),
        out_shape=jax.ShapeDtypeStruct((1, G), jnp.float32),
        scratch_shapes=[pltpu.VMEM((2, G), jnp.float32)],
    )(s, u, degp, b_conv2, W_read2, batch3)


_MESH = plsc.VectorSubcoreMesh(core_axis_name="c", subcore_axis_name="s")
NSUB = 16
RPT = NP // NSUB
CH = 64
NCH = 168
EC = NCH * CH
E2 = 32 * EC
NW = 2 * NSUB
NSEG = 7
SEG = NCH // NSEG
NSL = 3


@functools.partial(
    pl.kernel,
    mesh=_MESH,
    out_type=jax.ShapeDtypeStruct((2 * NP, DW), jnp.float32),
    scratch_types=[
        pltpu.VMEM((NCH, CH), jnp.int32),
        pltpu.VMEM((CH, DW), jnp.float32),
        pltpu.VMEM((CH, DW), jnp.float32),
        pltpu.VMEM_SHARED((NP, DW), jnp.float32),
    ],
)
def _sc_deg(dst_hbm, out_hbm, dst_buf, ones_v, zeros_v, acc_sh):
    c = lax.axis_index("c")
    t = lax.axis_index("s")
    wid = c * NSUB + t

    def _fill(r, carry):
        for j in range(DW // 16):
            ones_v[r, pl.ds(j * 16, 16)] = jnp.ones((16,), jnp.float32)
            zeros_v[r, pl.ds(j * 16, 16)] = jnp.zeros((16,), jnp.float32)
        return carry

    lax.fori_loop(0, CH, _fill, 0)
    pltpu.sync_copy(dst_hbm.at[pl.ds(wid * NCH, NCH)], dst_buf)
    for b in range(RPT // CH):
        pltpu.sync_copy(zeros_v, acc_sh.at[pl.ds(t * RPT + b * CH, CH)])
    plsc.subcore_barrier()

    def _step(k, carry):
        pltpu.sync_copy(ones_v, acc_sh.at[dst_buf.at[k]], add=True)
        return carry

    lax.fori_loop(0, NCH, _step, 0)
    plsc.subcore_barrier()
    pltpu.sync_copy(acc_sh.at[pl.ds(t * RPT, RPT)],
                    out_hbm.at[pl.ds(c * NP + t * RPT, RPT)])


@functools.partial(
    pl.kernel,
    mesh=_MESH,
    out_type=jax.ShapeDtypeStruct((2 * NPL * NP, PW), jnp.float32),
    scratch_types=(
        [pltpu.VMEM((SEG, CH), jnp.int32)] * 2
        + [pltpu.VMEM((CH,), jnp.int32)] * (2 * NSL)
        + [pltpu.VMEM((CH, PW), jnp.float32)] * NSL
        + [pltpu.VMEM_SHARED((NP, PW), jnp.float32)]
        + [pltpu.SemaphoreType.DMA] * (2 * NSL)
    ),
)
def _sc_scatter(u_hbm, src_hbm, dst_hbm, out_hbm, sbuf, dbuf,
                ps0, ps1, ps2, dv0, dv1, dv2, st0, st1, st2, acc_sh,
                gs0, gs1, gs2, ss0, ss1, ss2):
    c = lax.axis_index("c")
    t = lax.axis_index("s")
    wid = c * NSUB + t
    ps = [ps0, ps1, ps2]
    dv = [dv0, dv1, dv2]
    st = [st0, st1, st2]
    gs = [gs0, gs1, gs2]
    ss = [ss0, ss1, ss2]

    def _zrow(r, carry):
        for j in range(PW // 16):
            st0[r, pl.ds(j * 16, 16)] = jnp.zeros((16,), jnp.float32)
        return carry

    def _wait_s(r):
        pltpu.make_async_copy(st[r], acc_sh.at[dv[r]], ss[r]).wait()

    def _wait_g(r):
        pltpu.make_async_copy(u_hbm.at[ps[r]], st[r], gs[r]).wait()

    for p in range(NPL):
        lax.fori_loop(0, CH, _zrow, 0)
        for b in range(RPT // CH):
            pltpu.sync_copy(st0, acc_sh.at[pl.ds(t * RPT + b * CH, CH)])
        plsc.subcore_barrier()

        offv = jnp.broadcast_to(p * NP, (16,)).astype(jnp.int32)

        def _mk(r, k):
            for j in range(CH // 16):
                ps[r][pl.ds(j * 16, 16)] = sbuf[k, pl.ds(j * 16, 16)] + offv
                dv[r][pl.ds(j * 16, 16)] = dbuf[k, pl.ds(j * 16, 16)]

        for seg in range(NSEG):
            base = wid * NCH + seg * SEG
            pltpu.sync_copy(src_hbm.at[pl.ds(base, SEG)], sbuf)
            pltpu.sync_copy(dst_hbm.at[pl.ds(base, SEG)], dbuf)

            def _triple(g, carry):
                for r in range(NSL):
                    k = NSL * g + r
                    if seg == 0:
                        @pl.when(g > 0)
                        def _drain():
                            _wait_s(r)
                    else:
                        _wait_s(r)
                    _mk(r, k)
                    pltpu.async_copy(u_hbm.at[ps[r]], st[r], gs[r])
                for r in range(NSL):
                    _wait_g(r)
                    pltpu.async_copy(st[r], acc_sh.at[dv[r]], ss[r],
                                     add=True)
                return carry

            lax.fori_loop(0, SEG // NSL, _triple, 0)

        for r in range(NSL):
            _wait_s(r)
        plsc.subcore_barrier()
        pltpu.sync_copy(
            acc_sh.at[pl.ds(t * RPT, RPT)],
            out_hbm.at[pl.ds((c * NPL + p) * NP + t * RPT, RPT)],
        )


def kernel(x, edge_index, batch, W_pre, b_pre, W_conv, b_conv, W_read, b_read):
    pad = E2 - E
    src_pad = jnp.arange(pad, dtype=jnp.int32) % N
    src_p = jnp.concatenate([edge_index[0], src_pad])
    dst_pad = N + (jnp.arange(pad, dtype=jnp.int32) % (NP - N))
    dst_p = jnp.concatenate([edge_index[1], dst_pad])
    src4 = src_p.reshape(32 * NCH, CH)
    dst4 = dst_p.reshape(32 * NCH, CH)
    x_p = jnp.pad(x, ((0, NP - N), (0, 0)))
    batch3 = jnp.pad(batch, (0, NP - N), constant_values=G).reshape(NB, 1, BM)
    b_pre2 = b_pre.reshape(1, H)
    b_conv2 = b_conv.reshape(1, H)
    W_read2 = W_read.reshape(1, H)

    degp = _sc_deg(dst4).reshape(2, NP, DW)
    u = _tc_a(x_p, W_pre, b_pre2, W_conv, degp)
    for _ in range(2):
        s = _sc_scatter(u.reshape(NPL * NP, PW), src4, dst4).reshape(2, NPL, NP, PW)
        u = _tc_c(s, u, degp, b_conv2, W_conv)
    s = _sc_scatter(u.reshape(NPL * NP, PW), src4, dst4).reshape(2, NPL, NP, PW)
    out = _tc_d(s, u, degp, b_conv2, W_read2, batch3)
    return out.reshape(G) + b_read[0]
